# baseline (speedup 1.0000x reference)
import functools

import jax
import jax.numpy as jnp
from jax import lax
from jax.experimental import pallas as pl
from jax.experimental.pallas import tpu as pltpu
from jax.experimental.pallas import tpu_sc as plsc

_L = 16


@functools.lru_cache(maxsize=None)
def _build_sc_call(B, N, A, C, AA, SC_ATOMS):
    BB = A - SC_ATOMS
    P = A * C
    PBB = BB * C
    PSC = SC_ATOMS * C
    HALF = (SC_ATOMS + 1) // 2
    try:
        info = plsc.get_sparse_core_info()
        NC, NS = info.num_cores, info.num_subcores
    except ValueError:
        NC, NS = 2, 16
    mesh = plsc.VectorSubcoreMesh(
        core_axis_name="c", subcore_axis_name="s", num_cores=NC, num_subcores=NS
    )
    NW = NC * NS
    ROWS, BCOLS = 8, 128
    NB = B * N // (NW * ROWS * BCOLS)
    GROUPS = ROWS * BCOLS // _L
    GPR = BCOLS // _L
    WB = B // ROWS
    CW = NB * BCOLS

    @functools.partial(
        pl.kernel,
        out_type=jax.ShapeDtypeStruct((P, B, N), jnp.float32),
        mesh=mesh,
        scratch_types=[
            pltpu.VMEM((2, P, ROWS, BCOLS), jnp.float32),
            pltpu.VMEM((PSC, ROWS, BCOLS), jnp.float32),
            pltpu.VMEM((ROWS, CW), jnp.int32),
            pltpu.VMEM((AA * 2,), jnp.int32),
            pltpu.SemaphoreType.DMA,
            pltpu.SemaphoreType.DMA,
            pltpu.SemaphoreType.DMA,
            pltpu.SemaphoreType.DMA,
            pltpu.SemaphoreType.DMA,
            pltpu.SemaphoreType.DMA,
        ],
        compiler_params=pltpu.CompilerParams(
            needs_layout_passes=False,
            disable_bounds_checks=True,
            skip_device_barrier=True,
        ),
    )
    def sc_call(x_hbm, s_hbm, ptbl_hbm, out_hbm, xin, xout, sv, ptv,
                sem_in0, sem_in1, sem_s, sem_t, sem_osc, sem_obb):
        wid = lax.axis_index("s") * NC + lax.axis_index("c")
        r0 = (wid % WB) * ROWS
        c0 = (wid // WB) * CW
        sem_in = (sem_in0, sem_in1)

        def in_dma(b):
            return pltpu.make_async_copy(
                x_hbm.at[:, pl.ds(r0, ROWS), pl.ds(c0 + b * BCOLS, BCOLS)],
                xin.at[b], sem_in[b])

        def out_sc_dma(b):
            return pltpu.make_async_copy(
                xout,
                out_hbm.at[pl.ds(PBB, PSC), pl.ds(r0, ROWS),
                           pl.ds(c0 + b * BCOLS, BCOLS)],
                sem_osc)

        def out_bb_dma(b):
            return pltpu.make_async_copy(
                xin.at[b, pl.ds(0, PBB)],
                out_hbm.at[pl.ds(0, PBB), pl.ds(r0, ROWS),
                           pl.ds(c0 + b * BCOLS, BCOLS)],
                sem_obb)

        in_dma(0).start()
        cp_s = pltpu.make_async_copy(
            s_hbm.at[pl.ds(r0, ROWS), pl.ds(c0, CW)], sv, sem_s)
        cp_s.start()
        cp_t = pltpu.make_async_copy(ptbl_hbm, ptv, sem_t)
        cp_t.start()
        in_dma(1).start()
        cp_s.wait()
        cp_t.wait()
        lanes = lax.iota(jnp.int32, _L)

        for b in range(NB):
            in_dma(b).wait()
            out_bb_dma(b).start()
            if b >= 1:
                out_sc_dma(b - 1).wait()

            @plsc.parallel_loop(0, GROUPS, unroll=1)
            def group(g, b=b):
                s = g // GPR
                l0 = (g % GPR) * _L
                svec = sv[s, pl.ds(b * BCOLS + l0, _L)]
                widx = svec * 2
                w0 = plsc.load_gather(ptv, [widx])
                w1 = plsc.load_gather(ptv, [widx + 1])
                svec_s = jnp.full((_L,), s, jnp.int32)
                lvec = l0 + lanes
                xin_b = xin.at[b]
                for a in range(SC_ATOMS):
                    w = w0 if a < HALF else w1
                    p = ((w >> (6 * (a % HALF))) & 63) + PBB
                    for c in range(C):
                        v = plsc.load_gather(xin_b, [p + c, svec_s, lvec])
                        plsc.store_scatter(
                            xout,
                            [jnp.full((_L,), a * C + c, jnp.int32), svec_s,
                             lvec], v)

            out_sc_dma(b).start()

        out_sc_dma(NB - 1).wait()
        for b in range(NB):
            out_bb_dma(b).wait()

    return sc_call


def kernel(X, S, symmetry_indices):
    B, N, A, C = X.shape
    AA, SC_ATOMS = symmetry_indices.shape
    HALF = (SC_ATOMS + 1) // 2
    x_planes = jnp.transpose(X, (2, 3, 0, 1)).reshape(A * C, B, N)
    p3 = symmetry_indices.astype(jnp.int32) * C
    shifts = jnp.arange(HALF, dtype=jnp.int32) * 6
    w0 = jnp.sum(p3[:, :HALF] << shifts[None, :], axis=1)
    w1 = jnp.sum(p3[:, HALF:] << shifts[None, : SC_ATOMS - HALF], axis=1)
    ptbl = jnp.stack([w0, w1], axis=1).reshape(-1)
    sc_call = _build_sc_call(B, N, A, C, AA, SC_ATOMS)
    out_planes = sc_call(x_planes, S, ptbl)
    return jnp.transpose(out_planes.reshape(A, C, B, N), (2, 3, 0, 1))


# speedup vs baseline: 1.0454x; 1.0454x over previous
import functools

import jax
import jax.numpy as jnp
from jax import lax
from jax.experimental import pallas as pl
from jax.experimental.pallas import tpu as pltpu
from jax.experimental.pallas import tpu_sc as plsc

_L = 16


@functools.lru_cache(maxsize=None)
def _build_sc_call(B, N, A, C, AA, SC_ATOMS):
    BB = A - SC_ATOMS
    P = A * C
    PBB = BB * C
    PSC = SC_ATOMS * C
    HALF = (SC_ATOMS + 1) // 2
    try:
        info = plsc.get_sparse_core_info()
        NC, NS = info.num_cores, info.num_subcores
    except ValueError:
        NC, NS = 2, 16
    mesh = plsc.VectorSubcoreMesh(
        core_axis_name="c", subcore_axis_name="s", num_cores=NC, num_subcores=NS
    )
    NW = NC * NS
    ROWS, BCOLS = 8, 128
    NB = B * N // (NW * ROWS * BCOLS)
    assert NB == 2, "piece ping-pong below assumes exactly two pieces"
    GROUPS = ROWS * BCOLS // _L
    GPR = BCOLS // _L
    WB = B // ROWS
    CW = NB * BCOLS

    @functools.partial(
        pl.kernel,
        out_type=jax.ShapeDtypeStruct((P, B, N), jnp.float32),
        mesh=mesh,
        scratch_types=[
            pltpu.VMEM((P, ROWS, BCOLS), jnp.float32),
            pltpu.VMEM((P, ROWS, BCOLS), jnp.float32),
            pltpu.VMEM((PSC, ROWS, BCOLS), jnp.float32),
            pltpu.VMEM((ROWS, CW), jnp.int32),
            pltpu.VMEM((AA * 2,), jnp.int32),
            pltpu.SemaphoreType.DMA,
            pltpu.SemaphoreType.DMA,
            pltpu.SemaphoreType.DMA,
            pltpu.SemaphoreType.DMA,
            pltpu.SemaphoreType.DMA,
            pltpu.SemaphoreType.DMA,
        ],
        compiler_params=pltpu.CompilerParams(needs_layout_passes=False),
    )
    def sc_call(x_hbm, s_hbm, ptbl_hbm, out_hbm, xin0, xin1, xout, sv, ptv,
                sem_in0, sem_in1, sem_s, sem_t, sem_osc, sem_obb):
        wid = lax.axis_index("s") * NC + lax.axis_index("c")
        r0 = (wid % WB) * ROWS
        c0 = (wid // WB) * CW
        sem_in = (sem_in0, sem_in1)
        xin = (xin0, xin1)
        dst = (xout, xin0.at[pl.ds(PBB, PSC)])

        def in_dma(b):
            return pltpu.make_async_copy(
                x_hbm.at[:, pl.ds(r0, ROWS), pl.ds(c0 + b * BCOLS, BCOLS)],
                xin[b], sem_in[b])

        def out_sc_dma(b):
            return pltpu.make_async_copy(
                dst[b],
                out_hbm.at[pl.ds(PBB, PSC), pl.ds(r0, ROWS),
                           pl.ds(c0 + b * BCOLS, BCOLS)],
                sem_osc)

        def out_bb_dma(b):
            return pltpu.make_async_copy(
                xin[b].at[pl.ds(0, PBB)],
                out_hbm.at[pl.ds(0, PBB), pl.ds(r0, ROWS),
                           pl.ds(c0 + b * BCOLS, BCOLS)],
                sem_obb)

        in_dma(0).start()
        cp_s = pltpu.make_async_copy(
            s_hbm.at[pl.ds(r0, ROWS), pl.ds(c0, CW)], sv, sem_s)
        cp_s.start()
        cp_t = pltpu.make_async_copy(ptbl_hbm, ptv, sem_t)
        cp_t.start()
        in_dma(1).start()
        cp_s.wait()
        cp_t.wait()
        lanes = lax.iota(jnp.int32, _L)

        for b in range(NB):
            in_dma(b).wait()
            out_bb_dma(b).start()

            @plsc.parallel_loop(0, GROUPS, unroll=1)
            def group(g, b=b):
                s = g // GPR
                l0 = (g % GPR) * _L
                svec = sv[s, pl.ds(b * BCOLS + l0, _L)]
                widx = svec * 2
                w0 = plsc.load_gather(ptv, [widx])
                w1 = plsc.load_gather(ptv, [widx + 1])
                svec_s = jnp.full((_L,), s, jnp.int32)
                lvec = l0 + lanes
                for a in range(SC_ATOMS):
                    w = w0 if a < HALF else w1
                    p = ((w >> (6 * (a % HALF))) & 63) + PBB
                    for c in range(C):
                        v = plsc.load_gather(xin[b], [p + c, svec_s, lvec])
                        plsc.store_scatter(
                            dst[b],
                            [jnp.full((_L,), a * C + c, jnp.int32), svec_s,
                             lvec], v)

            out_sc_dma(b).start()

        for b in range(NB):
            out_sc_dma(b).wait()
            out_bb_dma(b).wait()

    return sc_call


def kernel(X, S, symmetry_indices):
    B, N, A, C = X.shape
    AA, SC_ATOMS = symmetry_indices.shape
    HALF = (SC_ATOMS + 1) // 2
    x_planes = jnp.transpose(X, (2, 3, 0, 1)).reshape(A * C, B, N)
    p3 = symmetry_indices.astype(jnp.int32) * C
    shifts = jnp.arange(HALF, dtype=jnp.int32) * 6
    w0 = jnp.sum(p3[:, :HALF] << shifts[None, :], axis=1)
    w1 = jnp.sum(p3[:, HALF:] << shifts[None, : SC_ATOMS - HALF], axis=1)
    ptbl = jnp.stack([w0, w1], axis=1).reshape(-1)
    sc_call = _build_sc_call(B, N, A, C, AA, SC_ATOMS)
    out_planes = sc_call(x_planes, S, ptbl)
    return jnp.transpose(out_planes.reshape(A, C, B, N), (2, 3, 0, 1))


# speedup vs baseline: 1.0659x; 1.0196x over previous
import functools

import jax
import jax.numpy as jnp
from jax import lax
from jax.experimental import pallas as pl
from jax.experimental.pallas import tpu as pltpu
from jax.experimental.pallas import tpu_sc as plsc

_L = 16


@functools.lru_cache(maxsize=None)
def _build_sc_call(B, N, A, C, AA, SC_ATOMS):
    BB = A - SC_ATOMS
    P = A * C
    PBB = BB * C
    PSC = SC_ATOMS * C
    HALF = (SC_ATOMS + 1) // 2
    try:
        info = plsc.get_sparse_core_info()
        NC, NS = info.num_cores, info.num_subcores
    except ValueError:
        NC, NS = 2, 16
    mesh = plsc.VectorSubcoreMesh(
        core_axis_name="c", subcore_axis_name="s", num_cores=NC, num_subcores=NS
    )
    NW = NC * NS
    ROWS, BCOLS = 8, 128
    NB = B * N // (NW * ROWS * BCOLS)
    assert NB == 2, "piece ping-pong below assumes exactly two pieces"
    GROUPS = ROWS * BCOLS // _L
    GPR = BCOLS // _L
    WB = B // ROWS
    CW = NB * BCOLS

    @functools.partial(
        pl.kernel,
        out_type=jax.ShapeDtypeStruct((P, B, N), jnp.float32),
        mesh=mesh,
        scratch_types=[
            pltpu.VMEM((P, ROWS, BCOLS), jnp.float32),
            pltpu.VMEM((P, ROWS, BCOLS), jnp.float32),
            pltpu.VMEM((PSC, ROWS, BCOLS), jnp.float32),
            pltpu.VMEM((ROWS, CW), jnp.int32),
            pltpu.VMEM((AA * 2,), jnp.int32),
            pltpu.SemaphoreType.DMA,
            pltpu.SemaphoreType.DMA,
            pltpu.SemaphoreType.DMA,
            pltpu.SemaphoreType.DMA,
            pltpu.SemaphoreType.DMA,
            pltpu.SemaphoreType.DMA,
            pltpu.SemaphoreType.DMA,
        ],
        compiler_params=pltpu.CompilerParams(needs_layout_passes=False),
    )
    def sc_call(x_hbm, s_hbm, ptbl_hbm, out_hbm, xin0, xin1, xout, sv, ptv,
                sem_in0, sem_in1, sem_ibb0, sem_s, sem_t, sem_osc, sem_obb):
        wid = lax.axis_index("s") * NC + lax.axis_index("c")
        r0 = (wid % WB) * ROWS
        c0 = (wid // WB) * CW
        sem_in = (sem_in0, sem_in1)
        xin = (xin0, xin1)
        dst = (xout, xin0.at[pl.ds(PBB, PSC)])

        def in_dma(b):
            return pltpu.make_async_copy(
                x_hbm.at[:, pl.ds(r0, ROWS), pl.ds(c0 + b * BCOLS, BCOLS)],
                xin[b], sem_in[b])

        def in_sc0_dma():
            return pltpu.make_async_copy(
                x_hbm.at[pl.ds(PBB, PSC), pl.ds(r0, ROWS),
                         pl.ds(c0, BCOLS)],
                xin0.at[pl.ds(PBB, PSC)], sem_in0)

        def in_bb0_dma():
            return pltpu.make_async_copy(
                x_hbm.at[pl.ds(0, PBB), pl.ds(r0, ROWS), pl.ds(c0, BCOLS)],
                xin0.at[pl.ds(0, PBB)], sem_ibb0)

        def out_sc_dma(b):
            return pltpu.make_async_copy(
                dst[b],
                out_hbm.at[pl.ds(PBB, PSC), pl.ds(r0, ROWS),
                           pl.ds(c0 + b * BCOLS, BCOLS)],
                sem_osc)

        def out_bb_dma(b):
            return pltpu.make_async_copy(
                xin[b].at[pl.ds(0, PBB)],
                out_hbm.at[pl.ds(0, PBB), pl.ds(r0, ROWS),
                           pl.ds(c0 + b * BCOLS, BCOLS)],
                sem_obb)

        in_sc0_dma().start()
        cp_s = pltpu.make_async_copy(
            s_hbm.at[pl.ds(r0, ROWS), pl.ds(c0, CW)], sv, sem_s)
        cp_s.start()
        cp_t = pltpu.make_async_copy(ptbl_hbm, ptv, sem_t)
        cp_t.start()
        in_bb0_dma().start()
        in_dma(1).start()
        cp_s.wait()
        cp_t.wait()
        lanes = lax.iota(jnp.int32, _L)

        for b in range(NB):
            if b == 0:
                in_sc0_dma().wait()
            else:
                in_dma(b).wait()
                out_bb_dma(b).start()

            @plsc.parallel_loop(0, GROUPS, unroll=1)
            def group(g, b=b):
                s = g // GPR
                l0 = (g % GPR) * _L
                svec = sv[s, pl.ds(b * BCOLS + l0, _L)]
                widx = svec * 2
                w0 = plsc.load_gather(ptv, [widx])
                w1 = plsc.load_gather(ptv, [widx + 1])
                svec_s = jnp.full((_L,), s, jnp.int32)
                lvec = l0 + lanes
                for a in range(SC_ATOMS):
                    w = w0 if a < HALF else w1
                    p = ((w >> (6 * (a % HALF))) & 63) + PBB
                    for c in range(C):
                        v = plsc.load_gather(xin[b], [p + c, svec_s, lvec])
                        plsc.store_scatter(
                            dst[b],
                            [jnp.full((_L,), a * C + c, jnp.int32), svec_s,
                             lvec], v)

            if b == 0:
                in_bb0_dma().wait()
                out_bb_dma(0).start()
            out_sc_dma(b).start()

        for b in range(NB):
            out_sc_dma(b).wait()
            out_bb_dma(b).wait()

    return sc_call


def kernel(X, S, symmetry_indices):
    B, N, A, C = X.shape
    AA, SC_ATOMS = symmetry_indices.shape
    HALF = (SC_ATOMS + 1) // 2
    x_planes = jnp.transpose(X, (2, 3, 0, 1)).reshape(A * C, B, N)
    p3 = symmetry_indices.astype(jnp.int32) * C
    shifts = jnp.arange(HALF, dtype=jnp.int32) * 6
    w0 = jnp.sum(p3[:, :HALF] << shifts[None, :], axis=1)
    w1 = jnp.sum(p3[:, HALF:] << shifts[None, : SC_ATOMS - HALF], axis=1)
    ptbl = jnp.stack([w0, w1], axis=1).reshape(-1)
    sc_call = _build_sc_call(B, N, A, C, AA, SC_ATOMS)
    out_planes = sc_call(x_planes, S, ptbl)
    return jnp.transpose(out_planes.reshape(A, C, B, N), (2, 3, 0, 1))
